# baseline (speedup 1.0000x reference)
import functools

import jax
import jax.numpy as jnp
from jax import lax
from jax.experimental import pallas as pl
from jax.experimental.pallas import tpu as pltpu
from jax.experimental.pallas import tpu_sc as plsc

NUM_WORKERS = 32
GATHER_W = 128


NBUF = 2


def _gather_sum(E_pad, L, C, D, idx2d, table, ca, cb):
    assert 16 * C * (ca + cb) == E_pad and (C * L) % GATHER_W == 0
    assert ca % NBUF == 0 and cb % NBUF == 0
    n_gathers = (C * L) // GATHER_W
    idx_rows_chunk = (C * L) // 128
    lanes = 16

    mesh = plsc.VectorSubcoreMesh(core_axis_name="c", subcore_axis_name="s")

    @functools.partial(
        pl.kernel,
        mesh=mesh,
        out_type=jax.ShapeDtypeStruct((E_pad, D), jnp.float32),
        scratch_types=(
            [pltpu.VMEM((idx_rows_chunk, 128), jnp.int32)] * NBUF
            + [pltpu.VMEM((C * L, D), jnp.float32)] * NBUF
            + [pltpu.VMEM((C, D), jnp.float32)] * NBUF
            + [pltpu.SemaphoreType.DMA] * (3 * NBUF)
        ),
    )
    def k(idx_hbm, tab_hbm, out_hbm, *bufs):
        idx_v = bufs[0:NBUF]
        rows_v = bufs[NBUF:2 * NBUF]
        out_v = bufs[2 * NBUF:3 * NBUF]
        sem_i = bufs[3 * NBUF:4 * NBUF]
        sem_g = bufs[4 * NBUF:5 * NBUF]
        sem_o = bufs[5 * NBUF:6 * NBUF]

        c = lax.axis_index("c")
        s = lax.axis_index("s")
        n_chunks = jnp.where(c == 0, ca, cb)
        chunk_base = jnp.where(c == 0, s * ca, 16 * ca + s * cb)
        row_base = chunk_base * C
        idx_row_base = chunk_base * idx_rows_chunk

        def idx_copy(ci, b):
            return pltpu.make_async_copy(
                idx_hbm.at[pl.ds(idx_row_base + ci * idx_rows_chunk,
                                 idx_rows_chunk)],
                idx_v[b], sem_i[b])

        def gather_copies(b):
            return [pltpu.make_async_copy(
                        tab_hbm.at[idx_v[b].at[g]],
                        rows_v[b].at[pl.ds(g * GATHER_W, GATHER_W)],
                        sem_g[b])
                    for g in range(n_gathers)]

        def out_copy(ci, b):
            return pltpu.make_async_copy(
                out_v[b], out_hbm.at[pl.ds(row_base + ci * C, C)], sem_o[b])

        def reduce_chunk(b):
            @pl.loop(0, C)
            def _(e):
                for f in range(D // lanes):
                    sl = pl.ds(f * lanes, lanes)
                    vals = [rows_v[b][e * L + l, sl] for l in range(L)]
                    while len(vals) > 1:
                        nxt = [vals[j] + vals[j + 1]
                               for j in range(0, len(vals) - 1, 2)]
                        if len(vals) % 2:
                            nxt.append(vals[-1])
                        vals = nxt
                    out_v[b][e, sl] = vals[0]

        for b in range(NBUF - 1):
            cp = idx_copy(b, b)
            cp.start()
            cp.wait()
            for g in gather_copies(b):
                g.start()
        idx_copy(NBUF - 1, NBUF - 1).start()

        @pl.loop(0, n_chunks, step=NBUF)
        def _(i0):
            for b in range(NBUF):
                ci = i0 + b
                fb = (b + NBUF - 1) % NBUF
                for g in gather_copies(b):
                    g.wait()

                @pl.when(ci + NBUF - 1 < n_chunks)
                def _():
                    idx_copy(ci + NBUF - 1, fb).wait()
                    for g in gather_copies(fb):
                        g.start()

                @pl.when(ci + NBUF < n_chunks)
                def _():
                    idx_copy(ci + NBUF, b).start()

                @pl.when(ci >= NBUF)
                def _():
                    out_copy(ci - NBUF, b).wait()

                reduce_chunk(b)
                out_copy(ci, b).start()

        for k_ in range(NBUF):
            out_copy(n_chunks - NBUF + k_, k_).wait()

    return k(idx2d, table)


def _stage2_body(L, g_ref, s_ref, x0_ref, w1_ref, w2_ref, o_ref):
    s = s_ref[...]
    m = (s > 0).astype(jnp.float32)
    c = jnp.sum(m, axis=1, keepdims=True)
    valid = c > 0
    u = jnp.where(valid, 1.0 / jnp.maximum(c, 1.0), 1.0 / L)
    z = jnp.where(valid, L - c, 0.0)
    edge_raw = (g_ref[...] - z * x0_ref[0:1, :]) * u
    edge = jnp.maximum(
        lax.dot(edge_raw, w1_ref[...], preferred_element_type=jnp.float32),
        0.0,
    )
    o_ref[...] = lax.dot(edge, w2_ref[...], preferred_element_type=jnp.float32)


def _stage2(g1, seq, x0t, W1, W2):
    E, L = seq.shape
    D = g1.shape[1]
    R = 1000
    assert E % R == 0
    return pl.pallas_call(
        functools.partial(_stage2_body, float(L)),
        grid=(E // R,),
        in_specs=[
            pl.BlockSpec((R, D), lambda i: (i, 0)),
            pl.BlockSpec((R, L), lambda i: (i, 0)),
            pl.BlockSpec((8, D), lambda i: (0, 0)),
            pl.BlockSpec((D, D), lambda i: (0, 0)),
            pl.BlockSpec((D, D), lambda i: (0, 0)),
        ],
        out_specs=pl.BlockSpec((R, D), lambda i: (i, 0)),
        out_shape=jax.ShapeDtypeStruct((E, D), jnp.float32),
    )(g1, seq, x0t, W1, W2)


def _stage3_body(L, g_ref, s_ref, e0_ref, o_ref):
    s = s_ref[...]
    m = (s > 0).astype(jnp.float32)
    c = jnp.sum(m, axis=1, keepdims=True)
    valid = c > 0
    u = jnp.where(valid, 1.0 / jnp.maximum(c, 1.0), 1.0 / L)
    z = jnp.where(valid, L - c, 0.0)
    o_ref[...] = (g_ref[...] - z * e0_ref[0:1, :]) * u


def _stage3(g2, useq, e0t):
    N, L = useq.shape
    D = g2.shape[1]
    R = 1000
    assert N % R == 0
    return pl.pallas_call(
        functools.partial(_stage3_body, float(L)),
        grid=(N // R,),
        in_specs=[
            pl.BlockSpec((R, D), lambda i: (i, 0)),
            pl.BlockSpec((R, L), lambda i: (i, 0)),
            pl.BlockSpec((8, D), lambda i: (0, 0)),
        ],
        out_specs=pl.BlockSpec((R, D), lambda i: (i, 0)),
        out_shape=jax.ShapeDtypeStruct((N, D), jnp.float32),
    )(g2, useq, e0t)


def _split(total_rows, C, frac0):
    n = -(-total_rows // (16 * C * NBUF)) * NBUF
    ca = max(NBUF, min(n - NBUF, int(round(n * frac0 / NBUF)) * NBUF))
    return ca, n - ca


@jax.jit
def kernel(x, seq, useq, W1, W2):
    E, L1 = seq.shape
    N, L2 = useq.shape
    D = x.shape[1]
    seq = seq.astype(jnp.int32)
    useq = useq.astype(jnp.int32)

    C1, C2 = 8, 16
    FRAC0 = 0.57
    ca1, cb1 = _split(E, C1, FRAC0)
    ca2, cb2 = _split(N, C2, FRAC0)
    E_pad = 16 * C1 * (ca1 + cb1)
    N_pad = 16 * C2 * (ca2 + cb2)

    idx1 = jnp.pad(seq.reshape(-1), (0, E_pad * L1 - E * L1)).reshape(-1, 128)
    idx2 = jnp.pad(useq.reshape(-1), (0, N_pad * L2 - N * L2)).reshape(-1, 128)

    g1 = _gather_sum(E_pad, L1, C1, D, idx1, x, ca1, cb1)
    e1 = _stage2(g1, seq, x, W1, W2)
    g2 = _gather_sum(N_pad, L2, C2, D, idx2, e1, ca2, cb2)
    return _stage3(g2, useq, e1)


# speedup vs baseline: 1.0654x; 1.0654x over previous
import functools

import jax
import jax.numpy as jnp
from jax import lax
from jax.experimental import pallas as pl
from jax.experimental.pallas import tpu as pltpu
from jax.experimental.pallas import tpu_sc as plsc

NUM_WORKERS = 32
GATHER_W = 128


NBUF = 2


def _gather_sum(E_pad, L, C, D, idx2d, table, ca, cb):
    assert 16 * C * (ca + cb) == E_pad and (C * L) % GATHER_W == 0
    assert ca % NBUF == 0 and cb % NBUF == 0
    n_gathers = (C * L) // GATHER_W
    idx_rows_chunk = (C * L) // 128
    lanes = 16

    mesh = plsc.VectorSubcoreMesh(core_axis_name="c", subcore_axis_name="s")

    @functools.partial(
        pl.kernel,
        mesh=mesh,
        out_type=jax.ShapeDtypeStruct((E_pad, D), jnp.float32),
        scratch_types=(
            [pltpu.VMEM((idx_rows_chunk, 128), jnp.int32)] * NBUF
            + [pltpu.VMEM((C * L, D), jnp.float32)] * NBUF
            + [pltpu.VMEM((C, D), jnp.float32)] * NBUF
            + [pltpu.SemaphoreType.DMA] * (3 * NBUF)
        ),
    )
    def k(idx_hbm, tab_hbm, out_hbm, *bufs):
        idx_v = bufs[0:NBUF]
        rows_v = bufs[NBUF:2 * NBUF]
        out_v = bufs[2 * NBUF:3 * NBUF]
        sem_i = bufs[3 * NBUF:4 * NBUF]
        sem_g = bufs[4 * NBUF:5 * NBUF]
        sem_o = bufs[5 * NBUF:6 * NBUF]

        c = lax.axis_index("c")
        s = lax.axis_index("s")
        n_chunks = jnp.where(c == 0, ca, cb)
        chunk_base = jnp.where(c == 0, s * ca, 16 * ca + s * cb)
        row_base = chunk_base * C
        idx_row_base = chunk_base * idx_rows_chunk

        def idx_copy(ci, b):
            return pltpu.make_async_copy(
                idx_hbm.at[pl.ds(idx_row_base + ci * idx_rows_chunk,
                                 idx_rows_chunk)],
                idx_v[b], sem_i[b])

        def gather_copies(b):
            return [pltpu.make_async_copy(
                        tab_hbm.at[idx_v[b].at[g]],
                        rows_v[b].at[pl.ds(g * GATHER_W, GATHER_W)],
                        sem_g[b])
                    for g in range(n_gathers)]

        def out_copy(ci, b):
            return pltpu.make_async_copy(
                out_v[b], out_hbm.at[pl.ds(row_base + ci * C, C)], sem_o[b])

        def reduce_chunk(b):
            @pl.loop(0, C)
            def _(e):
                for f in range(D // lanes):
                    sl = pl.ds(f * lanes, lanes)
                    vals = [rows_v[b][e * L + l, sl] for l in range(L)]
                    while len(vals) > 1:
                        nxt = [vals[j] + vals[j + 1]
                               for j in range(0, len(vals) - 1, 2)]
                        if len(vals) % 2:
                            nxt.append(vals[-1])
                        vals = nxt
                    out_v[b][e, sl] = vals[0]

        for b in range(NBUF - 1):
            cp = idx_copy(b, b)
            cp.start()
            cp.wait()
            for g in gather_copies(b):
                g.start()
        idx_copy(NBUF - 1, NBUF - 1).start()

        @pl.loop(0, n_chunks, step=NBUF)
        def _(i0):
            for b in range(NBUF):
                ci = i0 + b
                fb = (b + NBUF - 1) % NBUF
                for g in gather_copies(b):
                    g.wait()

                @pl.when(ci + NBUF - 1 < n_chunks)
                def _():
                    idx_copy(ci + NBUF - 1, fb).wait()
                    for g in gather_copies(fb):
                        g.start()

                @pl.when(ci + NBUF < n_chunks)
                def _():
                    idx_copy(ci + NBUF, b).start()

                @pl.when(ci >= NBUF)
                def _():
                    out_copy(ci - NBUF, b).wait()

                reduce_chunk(b)
                out_copy(ci, b).start()

        for k_ in range(NBUF):
            out_copy(n_chunks - NBUF + k_, k_).wait()

    return k(idx2d, table)


def _stage2_body(L, g_ref, s_ref, x0_ref, w1_ref, w2_ref, o_ref):
    s = s_ref[...]
    m = (s > 0).astype(jnp.float32)
    c = jnp.sum(m, axis=1, keepdims=True)
    valid = c > 0
    u = jnp.where(valid, 1.0 / jnp.maximum(c, 1.0), 1.0 / L)
    z = jnp.where(valid, L - c, 0.0)
    edge_raw = (g_ref[...] - z * x0_ref[0:1, :]) * u
    edge = jnp.maximum(
        lax.dot(edge_raw, w1_ref[...], preferred_element_type=jnp.float32),
        0.0,
    )
    o_ref[...] = lax.dot(edge, w2_ref[...], preferred_element_type=jnp.float32)


def _stage2(g1, seq, x0t, W1, W2):
    E, L = seq.shape
    D = g1.shape[1]
    R = 5000
    assert E % R == 0
    return pl.pallas_call(
        functools.partial(_stage2_body, float(L)),
        grid=(E // R,),
        in_specs=[
            pl.BlockSpec((R, D), lambda i: (i, 0)),
            pl.BlockSpec((R, L), lambda i: (i, 0)),
            pl.BlockSpec((8, D), lambda i: (0, 0)),
            pl.BlockSpec((D, D), lambda i: (0, 0)),
            pl.BlockSpec((D, D), lambda i: (0, 0)),
        ],
        out_specs=pl.BlockSpec((R, D), lambda i: (i, 0)),
        out_shape=jax.ShapeDtypeStruct((E, D), jnp.float32),
    )(g1, seq, x0t, W1, W2)


def _stage3_body(L, g_ref, s_ref, e0_ref, o_ref):
    s = s_ref[...]
    m = (s > 0).astype(jnp.float32)
    c = jnp.sum(m, axis=1, keepdims=True)
    valid = c > 0
    u = jnp.where(valid, 1.0 / jnp.maximum(c, 1.0), 1.0 / L)
    z = jnp.where(valid, L - c, 0.0)
    o_ref[...] = (g_ref[...] - z * e0_ref[0:1, :]) * u


def _stage3(g2, useq, e0t):
    N, L = useq.shape
    D = g2.shape[1]
    R = 5000
    assert N % R == 0
    return pl.pallas_call(
        functools.partial(_stage3_body, float(L)),
        grid=(N // R,),
        in_specs=[
            pl.BlockSpec((R, D), lambda i: (i, 0)),
            pl.BlockSpec((R, L), lambda i: (i, 0)),
            pl.BlockSpec((8, D), lambda i: (0, 0)),
        ],
        out_specs=pl.BlockSpec((R, D), lambda i: (i, 0)),
        out_shape=jax.ShapeDtypeStruct((N, D), jnp.float32),
    )(g2, useq, e0t)


def _split(total_rows, C, frac0):
    n = -(-total_rows // (16 * C * NBUF)) * NBUF
    ca = max(NBUF, min(n - NBUF, int(round(n * frac0 / NBUF)) * NBUF))
    return ca, n - ca


@jax.jit
def kernel(x, seq, useq, W1, W2):
    E, L1 = seq.shape
    N, L2 = useq.shape
    D = x.shape[1]
    seq = seq.astype(jnp.int32)
    useq = useq.astype(jnp.int32)

    C1, C2 = 8, 16
    FRAC0 = 0.57
    ca1, cb1 = _split(E, C1, FRAC0)
    ca2, cb2 = _split(N, C2, FRAC0)
    E_pad = 16 * C1 * (ca1 + cb1)
    N_pad = 16 * C2 * (ca2 + cb2)

    idx1 = jnp.pad(seq.reshape(-1), (0, E_pad * L1 - E * L1)).reshape(-1, 128)
    idx2 = jnp.pad(useq.reshape(-1), (0, N_pad * L2 - N * L2)).reshape(-1, 128)

    g1 = _gather_sum(E_pad, L1, C1, D, idx1, x, ca1, cb1)
    e1 = _stage2(g1, seq, x, W1, W2)
    g2 = _gather_sum(N_pad, L2, C2, D, idx2, e1, ca2, cb2)
    return _stage3(g2, useq, e1)
